# baseline (speedup 1.0000x reference)
import dataclasses
import functools

import jax
import jax.numpy as jnp
from jax import lax
from jax.experimental import pallas as pl
from jax.experimental.pallas import tpu as pltpu
from jax.experimental.pallas import tpu_sc as plsc

N = 10000
E = 320000
D_IN = 128
H = 16
D_OUT = 128

NT = 32
NB = 32
SH = 4
NB2 = NB * SH
SQ = 2560
NPB = 320
NPAD = NB * NPB
EW = E // NT
GROUPS = EW // 16
STG = 160
FLUSH = 144
NFLUSH_MAX = EW // FLUSH
CAPQ = NFLUSH_MAX * FLUSH + STG
K = 128
ROWBLK = 1000

_MAGIC = 13108
_MAGICQ = 3277


def _mesh():
    return plsc.VectorSubcoreMesh(core_axis_name="c", subcore_axis_name="s")


def _sc_params():
    cp = pltpu.CompilerParams()
    if "needs_layout_passes" in pltpu.CompilerParams.__dataclass_fields__:
        cp = dataclasses.replace(cp, needs_layout_passes=False)
    return cp


def _bucket_body(src_hbm, dst_hbm, w_hbm, qsrc, qdst, qw, cnt16_hbm,
                 ch_src, ch_dst, ch_w, stg_src, stg_dst, stg_w,
                 cnt_v, out16_v, hnum_s, sem):
    wid = lax.axis_index("s") * 2 + lax.axis_index("c")
    base = wid * EW
    iota = lax.iota(jnp.int32, 16)
    pr, _ = plsc.scan_count(jnp.zeros((16,), jnp.int32))
    rbase = pr - iota

    pltpu.sync_copy(src_hbm.at[pl.ds(base, EW)], ch_src)
    pltpu.sync_copy(dst_hbm.at[pl.ds(base, EW)], ch_dst)
    pltpu.sync_copy(w_hbm.at[pl.ds(base, EW)], ch_w)

    zeros16 = jnp.zeros((16,), jnp.int32)
    for z in range(NB2 // 16):
        cnt_v[pl.ds(z * 16, 16)] = zeros16

    @pl.loop(0, NB2)
    def _(b):
        hnum_s[b] = 0

    def _cnt_of(b):
        cv = plsc.load_gather(cnt_v, [jnp.full((16,), b, jnp.int32)])
        return jnp.max(cv)

    def _flush(b):
        cb = _cnt_of(b)

        @pl.when(cb >= FLUSH)
        def _():
            hn = hnum_s[b]
            off = (wid * NB2 + b) * CAPQ + hn * FLUSH
            sb = b * STG
            c1 = pltpu.async_copy(stg_src.at[pl.ds(sb, FLUSH)],
                                  qsrc.at[pl.ds(off, FLUSH)], sem)
            c2 = pltpu.async_copy(stg_dst.at[pl.ds(sb, FLUSH)],
                                  qdst.at[pl.ds(off, FLUSH)], sem)
            c3 = pltpu.async_copy(stg_w.at[pl.ds(sb, FLUSH)],
                                  qw.at[pl.ds(off, FLUSH)], sem)
            c1.wait()
            c2.wait()
            c3.wait()
            stg_src[pl.ds(sb, 16)] = stg_src[pl.ds(sb + FLUSH, 16)]
            stg_dst[pl.ds(sb, 16)] = stg_dst[pl.ds(sb + FLUSH, 16)]
            stg_w[pl.ds(sb, 16)] = stg_w[pl.ds(sb + FLUSH, 16)]
            hnum_s[b] = hn + 1
            bidx = jnp.full((16,), b, jnp.int32)
            plsc.store_scatter(cnt_v, [bidx], jnp.full((16,), cb - FLUSH,
                                                       jnp.int32))

    @pl.loop(0, GROUPS)
    def _(g):
        dv = ch_dst[pl.ds(g * 16, 16)]
        sv = ch_src[pl.ds(g * 16, 16)]
        wv = ch_w[pl.ds(g * 16, 16)]
        bv = (dv * _MAGIC) >> 22
        dloc = dv - bv * NPB
        bv2 = bv * SH + ((sv * _MAGICQ) >> 23)
        rank, last = plsc.scan_count(bv2)
        rank0 = rank - rbase
        cnts = plsc.load_gather(cnt_v, [bv2])
        addr = bv2 * STG + cnts + rank0
        plsc.store_scatter(stg_src, [addr], sv)
        plsc.store_scatter(stg_dst, [addr], dloc)
        plsc.store_scatter(stg_w, [addr], wv)
        newcnt = cnts + rank0 + 1
        plsc.store_scatter(cnt_v, [bv2], newcnt, mask=last)
        trig = jnp.max(jnp.where(last & (newcnt >= FLUSH), 1, 0))

        @pl.when(trig > 0)
        def _():
            @pl.loop(0, NB2)
            def _(b):
                _flush(b)

    @pl.loop(0, NB2)
    def _(b):
        cb = _cnt_of(b)
        hn = hnum_s[b]
        off = (wid * NB2 + b) * CAPQ + hn * FLUSH
        sb = b * STG
        blk = sb + ((cb >> 4) << 4)
        padm = iota >= (cb & 15)
        pad_src = jnp.full((16,), (b & (SH - 1)) * SQ, jnp.int32)
        z_i = jnp.zeros((16,), jnp.int32)
        z_f = jnp.zeros((16,), jnp.float32)
        stg_src[pl.ds(blk, 16)] = jnp.where(padm, pad_src,
                                            stg_src[pl.ds(blk, 16)])
        stg_dst[pl.ds(blk, 16)] = jnp.where(padm, z_i, stg_dst[pl.ds(blk, 16)])
        stg_w[pl.ds(blk, 16)] = jnp.where(padm, z_f, stg_w[pl.ds(blk, 16)])
        c1 = pltpu.async_copy(stg_src.at[pl.ds(sb, STG)],
                              qsrc.at[pl.ds(off, STG)], sem)
        c2 = pltpu.async_copy(stg_dst.at[pl.ds(sb, STG)],
                              qdst.at[pl.ds(off, STG)], sem)
        c3 = pltpu.async_copy(stg_w.at[pl.ds(sb, STG)],
                              qw.at[pl.ds(off, STG)], sem)
        out16_v[...] = jnp.full((16,), hn * FLUSH + cb, jnp.int32)
        c4 = pltpu.async_copy(out16_v,
                              cnt16_hbm.at[pl.ds((b * NT + wid) * 16, 16)],
                              sem)
        c1.wait()
        c2.wait()
        c3.wait()
        c4.wait()


def _bucket(src, dst, w):
    i32 = jnp.int32
    f32 = jnp.float32
    out_type = (
        jax.ShapeDtypeStruct((NT * NB2 * CAPQ,), i32),
        jax.ShapeDtypeStruct((NT * NB2 * CAPQ,), i32),
        jax.ShapeDtypeStruct((NT * NB2 * CAPQ,), f32),
        jax.ShapeDtypeStruct((NT * NB2 * 16,), i32),
    )
    scratch = [
        pltpu.VMEM((EW,), i32), pltpu.VMEM((EW,), i32), pltpu.VMEM((EW,), f32),
        pltpu.VMEM((NB2 * STG,), i32), pltpu.VMEM((NB2 * STG,), i32),
        pltpu.VMEM((NB2 * STG,), f32),
        pltpu.VMEM((NB2,), i32),
        pltpu.VMEM((16,), i32),
        pltpu.SMEM((NB2,), i32),
        pltpu.SemaphoreType.DMA,
    ]
    return pl.kernel(_bucket_body, out_type=out_type, mesh=_mesh(),
                     scratch_types=scratch,
                     compiler_params=_sc_params())(src, dst, w)


def _aggr16_body(h_hbm, qsrc, qdst, qw, cnt16_hbm, aggr_hbm,
                 table, acc0, acc1, acc2, acc3, src_a, dst_a, w_a,
                 src_b, dst_b, w_b, cblk_v, sem):
    t = lax.axis_index("s") * 2 + lax.axis_index("c")
    iota = lax.iota(jnp.int32, 16)
    HALF = 2 * SQ * H
    accs = [acc0, acc1, acc2, acc3]

    pltpu.sync_copy(cnt16_hbm.at[pl.ds(t * SH * NT * 16, SH * NT * 16)],
                    cblk_v)

    @pl.loop(0, NPB * H // 16)
    def _(z):
        zf = jnp.zeros((16,), jnp.float32)
        for a in accs:
            a[pl.ds(z * 16, 16)] = zf

    def _rmw16(src_v, dst_v, w_v, m, lo):
        mq = (m + 3) >> 2

        @pl.loop(0, mq)
        def _(i4):
            news = []
            for k in range(4):
                i = i4 * 4 + k
                fi = jnp.full((16,), i, jnp.int32)
                s = plsc.load_gather(src_v, [fi])
                dl = plsc.load_gather(dst_v, [fi])
                ww = plsc.load_gather(w_v, [fi])
                r = plsc.load_gather(table, [(s - lo) * H + iota])
                aj = dl * H + iota
                a = plsc.load_gather(accs[k], [aj])
                news.append((aj, jnp.maximum(a, r * ww)))
            for k in range(4):
                aj, nv = news[k]
                plsc.store_scatter(accs[k], [aj], nv)

    for half in range(2):
        lo = half * 2 * SQ
        pltpu.sync_copy(h_hbm.at[pl.ds(half * HALF, HALF)], table)

        for qq in (2 * half, 2 * half + 1):

            @pl.loop(0, NT)
            def _(w):
                slot = w * NB2 + t * SH + qq
                cnt = jnp.max(cblk_v[pl.ds((qq * NT + w) * 16, 16)])
                nch = (cnt + (K - 1)) >> 7

                @pl.loop(0, nch)
                def _(c):
                    off = slot * CAPQ + c * K
                    m = jnp.minimum(cnt - c * K, K)
                    c1 = pltpu.async_copy(qsrc.at[pl.ds(off, K)], src_a, sem)
                    c2 = pltpu.async_copy(qdst.at[pl.ds(off, K)], dst_a, sem)
                    c3 = pltpu.async_copy(qw.at[pl.ds(off, K)], w_a, sem)
                    c1.wait()
                    c2.wait()
                    c3.wait()
                    _rmw16(src_a, dst_a, w_a, m, lo)

    @pl.loop(0, NPB * H // 16)
    def _(z):
        sl = pl.ds(z * 16, 16)
        acc0[sl] = jnp.maximum(jnp.maximum(acc0[sl], acc1[sl]),
                               jnp.maximum(acc2[sl], acc3[sl]))

    pltpu.sync_copy(acc0, aggr_hbm.at[pl.ds(t * NPB * H, NPB * H)])


def _aggregate16(h1flat, qsrc, qdst, qw, cnt16):
    f32 = jnp.float32
    i32 = jnp.int32
    scratch = [
        pltpu.VMEM((2 * SQ * H,), f32),
        pltpu.VMEM((NPB * H,), f32),
        pltpu.VMEM((NPB * H,), f32),
        pltpu.VMEM((NPB * H,), f32),
        pltpu.VMEM((NPB * H,), f32),
        pltpu.VMEM((K,), i32),
        pltpu.VMEM((K,), i32),
        pltpu.VMEM((K,), f32),
        pltpu.VMEM((K,), i32),
        pltpu.VMEM((K,), i32),
        pltpu.VMEM((K,), f32),
        pltpu.VMEM((SH * NT * 16,), i32),
        pltpu.SemaphoreType.DMA,
    ]
    return pl.kernel(_aggr16_body,
                     out_type=jax.ShapeDtypeStruct((NPAD * H,), f32),
                     mesh=_mesh(), scratch_types=scratch,
                     compiler_params=_sc_params())(
                         h1flat, qsrc, qdst, qw, cnt16)


def _aggr_body(h_hbm, qsrc, qdst, qw, cnt16_hbm, aggr_hbm,
               hsh, acc0, acc1, rows_a, idx_a, dst_a, w_a,
               idx_b, dst_b, w_b, cblk_v, sem):
    D = D_OUT
    t = lax.axis_index("s") * 2 + lax.axis_index("c")
    iota = lax.iota(jnp.int32, 16)
    NV = D // 16
    accs = [acc0, acc1]

    pltpu.sync_copy(cnt16_hbm.at[pl.ds(t * SH * NT * 16, SH * NT * 16)],
                    cblk_v)

    @pl.loop(0, NPB * D // 16)
    def _(z):
        zf = jnp.zeros((16,), jnp.float32)
        acc0[pl.ds(z * 16, 16)] = zf
        acc1[pl.ds(z * 16, 16)] = zf

    def _sanitize(idx_v, m, lob):
        for v in range(K // 16):
            lanes = iota + v * 16
            cur = idx_v[pl.ds(v * 16, 16)] - lob
            idx_v[pl.ds(v * 16, 16)] = jnp.where(lanes < m, cur, 0)

    def _rmw(rows, dst_v, w_v, m):
        m2 = (m + 1) >> 1

        @pl.loop(0, m2)
        def _(i2):
            news = []
            for k in range(2):
                i = i2 * 2 + k
                fi = jnp.full((16,), i, jnp.int32)
                dl = plsc.load_gather(dst_v, [fi])
                ww = plsc.load_gather(w_v, [fi])
                base = dl * D
                for j in range(NV):
                    aj = base + (iota + j * 16)
                    a = plsc.load_gather(accs[k], [aj])
                    r = rows[i, pl.ds(j * 16, 16)]
                    news.append((k, aj, jnp.maximum(a, r * ww)))
            for k, aj, nv in news:
                plsc.store_scatter(accs[k], [aj], nv)

    for qq in range(SH):
        plsc.subcore_barrier()

        @pl.when(lax.axis_index("s") == 0)
        def _():
            pltpu.sync_copy(h_hbm.at[pl.ds(qq * SQ, SQ), :], hsh)

        plsc.subcore_barrier()
        lob = jnp.full((16,), qq * SQ, jnp.int32)

        @pl.loop(0, NT)
        def _(w):
            slot = w * NB2 + t * SH + qq
            cnt = jnp.max(cblk_v[pl.ds((qq * NT + w) * 16, 16)])
            nch = (cnt + (K - 1)) >> 7

            @pl.loop(0, nch)
            def _(c):
                off = slot * CAPQ + c * K
                m = jnp.minimum(cnt - c * K, K)
                c1 = pltpu.async_copy(qsrc.at[pl.ds(off, K)], idx_a, sem)
                c2 = pltpu.async_copy(qdst.at[pl.ds(off, K)], dst_a, sem)
                c3 = pltpu.async_copy(qw.at[pl.ds(off, K)], w_a, sem)
                c1.wait()
                _sanitize(idx_a, m, lob)
                pltpu.sync_copy(hsh.at[idx_a], rows_a)
                c2.wait()
                c3.wait()
                _rmw(rows_a, dst_a, w_a, m)

    @pl.loop(0, NPB * D // 16)
    def _(z):
        sl = pl.ds(z * 16, 16)
        acc0[sl] = jnp.maximum(acc0[sl], acc1[sl])

    pltpu.sync_copy(acc0, aggr_hbm.at[pl.ds(t * NPB * D, NPB * D)])


def _aggregate(h, qsrc, qdst, qw, cnt16):
    f32 = jnp.float32
    D = D_OUT
    i32 = jnp.int32
    scratch = [
        pltpu.VMEM_SHARED((SQ, D), f32),
        pltpu.VMEM((NPB * D,), f32),
        pltpu.VMEM((NPB * D,), f32),
        pltpu.VMEM((K, D), f32),
        pltpu.VMEM((K,), i32),
        pltpu.VMEM((K,), i32),
        pltpu.VMEM((K,), f32),
        pltpu.VMEM((K,), i32),
        pltpu.VMEM((K,), i32),
        pltpu.VMEM((K,), f32),
        pltpu.VMEM((SH * NT * 16,), i32),
        pltpu.SemaphoreType.DMA,
    ]
    return pl.kernel(_aggr_body,
                     out_type=jax.ShapeDtypeStruct((NPAD * D,), f32),
                     mesh=_mesh(), scratch_types=scratch,
                     compiler_params=_sc_params())(
                         h, qsrc, qdst, qw, cnt16)


def _mm1_body(x_ref, wt_ref, b_ref, o_ref):
    o_ref[...] = lax.dot_general(
        x_ref[...], wt_ref[...], (((1,), (0,)), ((), ())),
        preferred_element_type=jnp.float32) + b_ref[...]


def _mm1(x, w1t, b1):
    grid = N // ROWBLK
    return pl.pallas_call(
        _mm1_body,
        grid=(grid,),
        in_specs=[
            pl.BlockSpec((ROWBLK, D_IN), lambda i: (i, 0)),
            pl.BlockSpec((D_IN, H), lambda i: (0, 0)),
            pl.BlockSpec((1, H), lambda i: (0, 0)),
        ],
        out_specs=pl.BlockSpec((ROWBLK, H), lambda i: (i, 0)),
        out_shape=jax.ShapeDtypeStruct((N, H), jnp.float32),
    )(x, w1t, b1)


def _mid_body(h1_ref, a1_ref, wt_ref, b_ref, o_ref):
    x2 = jnp.maximum(h1_ref[...] - a1_ref[...], 0.0)
    o_ref[...] = lax.dot_general(
        x2, wt_ref[...], (((1,), (0,)), ((), ())),
        preferred_element_type=jnp.float32) + b_ref[...]


def _mid(h1, a1, w2t, b2):
    grid = N // ROWBLK
    return pl.pallas_call(
        _mid_body,
        grid=(grid,),
        in_specs=[
            pl.BlockSpec((ROWBLK, H), lambda i: (i, 0)),
            pl.BlockSpec((ROWBLK, H), lambda i: (i, 0)),
            pl.BlockSpec((H, D_OUT), lambda i: (0, 0)),
            pl.BlockSpec((1, D_OUT), lambda i: (0, 0)),
        ],
        out_specs=pl.BlockSpec((ROWBLK, D_OUT), lambda i: (i, 0)),
        out_shape=jax.ShapeDtypeStruct((N, D_OUT), jnp.float32),
    )(h1, a1, w2t, b2)


def _final_body(h2_ref, a2_ref, o_ref):
    z = h2_ref[...] - a2_ref[...]
    zmax = jnp.max(z, axis=1, keepdims=True)
    ez = jnp.exp(z - zmax)
    s = jnp.sum(ez, axis=1, keepdims=True)
    o_ref[...] = z - zmax - jnp.log(s)


def _final(h2, a2):
    grid = N // ROWBLK
    return pl.pallas_call(
        _final_body,
        grid=(grid,),
        in_specs=[
            pl.BlockSpec((ROWBLK, D_OUT), lambda i: (i, 0)),
            pl.BlockSpec((ROWBLK, D_OUT), lambda i: (i, 0)),
        ],
        out_specs=pl.BlockSpec((ROWBLK, D_OUT), lambda i: (i, 0)),
        out_shape=jax.ShapeDtypeStruct((N, D_OUT), jnp.float32),
    )(h2, a2)


def kernel(x, edge_index, edge_weight, W1, b1, W2, b2):
    src = edge_index[0]
    dst = edge_index[1]
    qsrc, qdst, qw, cnt16 = _bucket(src, dst, edge_weight)
    h1 = _mm1(x, W1.T, b1.reshape(1, H))
    h1p = jnp.pad(h1, ((0, SH * SQ - N), (0, 0)))
    a1 = _aggregate16(h1p.reshape(SH * SQ * H), qsrc, qdst, qw, cnt16)
    a1 = a1.reshape(NPAD, H)[:N]
    h2 = _mid(h1, a1, W2.T, b2.reshape(1, D_OUT))
    h2p = jnp.pad(h2, ((0, SH * SQ - N), (0, 0)))
    a2 = _aggregate(h2p, qsrc, qdst, qw, cnt16)
    a2 = a2.reshape(NPAD, D_OUT)[:N]
    return _final(h2, a2)


# speedup vs baseline: 1.1073x; 1.1073x over previous
import dataclasses
import functools

import jax
import jax.numpy as jnp
from jax import lax
from jax.experimental import pallas as pl
from jax.experimental.pallas import tpu as pltpu
from jax.experimental.pallas import tpu_sc as plsc

N = 10000
E = 320000
D_IN = 128
H = 16
D_OUT = 128

NT = 32
NB = 32
SH = 4
NB2 = NB * SH
SQ = 2560
NPB = 320
NPAD = NB * NPB
EW = E // NT
GROUPS = EW // 16
STG = 160
FLUSH = 144
NFLUSH_MAX = EW // FLUSH
CAPQ = NFLUSH_MAX * FLUSH + STG
K = 128
ROWBLK = 1000

_MAGIC = 13108
_MAGICQ = 3277


def _mesh():
    return plsc.VectorSubcoreMesh(core_axis_name="c", subcore_axis_name="s")


def _sc_params():
    cp = pltpu.CompilerParams()
    if "needs_layout_passes" in pltpu.CompilerParams.__dataclass_fields__:
        cp = dataclasses.replace(cp, needs_layout_passes=False)
    return cp


def _bucket_body(src_hbm, dst_hbm, w_hbm, qsrc, qdst, qw, cnt16_hbm,
                 ch_src, ch_dst, ch_w, stg_src, stg_dst, stg_w,
                 cnt_v, out16_v, hnum_s, sem):
    wid = lax.axis_index("s") * 2 + lax.axis_index("c")
    base = wid * EW
    iota = lax.iota(jnp.int32, 16)
    pr, _ = plsc.scan_count(jnp.zeros((16,), jnp.int32))
    rbase = pr - iota

    pltpu.sync_copy(src_hbm.at[pl.ds(base, EW)], ch_src)
    pltpu.sync_copy(dst_hbm.at[pl.ds(base, EW)], ch_dst)
    pltpu.sync_copy(w_hbm.at[pl.ds(base, EW)], ch_w)

    zeros16 = jnp.zeros((16,), jnp.int32)
    for z in range(NB2 // 16):
        cnt_v[pl.ds(z * 16, 16)] = zeros16

    @pl.loop(0, NB2)
    def _(b):
        hnum_s[b] = 0

    def _cnt_of(b):
        cv = plsc.load_gather(cnt_v, [jnp.full((16,), b, jnp.int32)])
        return jnp.max(cv)

    def _flush(b):
        cb = _cnt_of(b)

        @pl.when(cb >= FLUSH)
        def _():
            hn = hnum_s[b]
            off = (wid * NB2 + b) * CAPQ + hn * FLUSH
            sb = b * STG
            c1 = pltpu.async_copy(stg_src.at[pl.ds(sb, FLUSH)],
                                  qsrc.at[pl.ds(off, FLUSH)], sem)
            c2 = pltpu.async_copy(stg_dst.at[pl.ds(sb, FLUSH)],
                                  qdst.at[pl.ds(off, FLUSH)], sem)
            c3 = pltpu.async_copy(stg_w.at[pl.ds(sb, FLUSH)],
                                  qw.at[pl.ds(off, FLUSH)], sem)
            c1.wait()
            c2.wait()
            c3.wait()
            stg_src[pl.ds(sb, 16)] = stg_src[pl.ds(sb + FLUSH, 16)]
            stg_dst[pl.ds(sb, 16)] = stg_dst[pl.ds(sb + FLUSH, 16)]
            stg_w[pl.ds(sb, 16)] = stg_w[pl.ds(sb + FLUSH, 16)]
            hnum_s[b] = hn + 1
            bidx = jnp.full((16,), b, jnp.int32)
            plsc.store_scatter(cnt_v, [bidx], jnp.full((16,), cb - FLUSH,
                                                       jnp.int32))

    @pl.loop(0, GROUPS)
    def _(g):
        dv = ch_dst[pl.ds(g * 16, 16)]
        sv = ch_src[pl.ds(g * 16, 16)]
        wv = ch_w[pl.ds(g * 16, 16)]
        bv = (dv * _MAGIC) >> 22
        dloc = dv - bv * NPB
        bv2 = bv * SH + ((sv * _MAGICQ) >> 23)
        rank, last = plsc.scan_count(bv2)
        rank0 = rank - rbase
        cnts = plsc.load_gather(cnt_v, [bv2])
        addr = bv2 * STG + cnts + rank0
        plsc.store_scatter(stg_src, [addr], sv)
        plsc.store_scatter(stg_dst, [addr], dloc)
        plsc.store_scatter(stg_w, [addr], wv)
        newcnt = cnts + rank0 + 1
        plsc.store_scatter(cnt_v, [bv2], newcnt, mask=last)
        trig = jnp.max(jnp.where(last & (newcnt >= FLUSH), 1, 0))

        @pl.when(trig > 0)
        def _():
            @pl.loop(0, NB2)
            def _(b):
                _flush(b)

    @pl.loop(0, NB2)
    def _(b):
        cb = _cnt_of(b)
        hn = hnum_s[b]
        off = (wid * NB2 + b) * CAPQ + hn * FLUSH
        sb = b * STG
        blk = sb + ((cb >> 4) << 4)
        padm = iota >= (cb & 15)
        pad_src = jnp.full((16,), (b & (SH - 1)) * SQ, jnp.int32)
        z_i = jnp.zeros((16,), jnp.int32)
        z_f = jnp.zeros((16,), jnp.float32)
        stg_src[pl.ds(blk, 16)] = jnp.where(padm, pad_src,
                                            stg_src[pl.ds(blk, 16)])
        stg_dst[pl.ds(blk, 16)] = jnp.where(padm, z_i, stg_dst[pl.ds(blk, 16)])
        stg_w[pl.ds(blk, 16)] = jnp.where(padm, z_f, stg_w[pl.ds(blk, 16)])
        c1 = pltpu.async_copy(stg_src.at[pl.ds(sb, STG)],
                              qsrc.at[pl.ds(off, STG)], sem)
        c2 = pltpu.async_copy(stg_dst.at[pl.ds(sb, STG)],
                              qdst.at[pl.ds(off, STG)], sem)
        c3 = pltpu.async_copy(stg_w.at[pl.ds(sb, STG)],
                              qw.at[pl.ds(off, STG)], sem)
        out16_v[...] = jnp.full((16,), hn * FLUSH + cb, jnp.int32)
        c4 = pltpu.async_copy(out16_v,
                              cnt16_hbm.at[pl.ds((b * NT + wid) * 16, 16)],
                              sem)
        c1.wait()
        c2.wait()
        c3.wait()
        c4.wait()


def _bucket(src, dst, w):
    i32 = jnp.int32
    f32 = jnp.float32
    out_type = (
        jax.ShapeDtypeStruct((NT * NB2 * CAPQ,), i32),
        jax.ShapeDtypeStruct((NT * NB2 * CAPQ,), i32),
        jax.ShapeDtypeStruct((NT * NB2 * CAPQ,), f32),
        jax.ShapeDtypeStruct((NT * NB2 * 16,), i32),
    )
    scratch = [
        pltpu.VMEM((EW,), i32), pltpu.VMEM((EW,), i32), pltpu.VMEM((EW,), f32),
        pltpu.VMEM((NB2 * STG,), i32), pltpu.VMEM((NB2 * STG,), i32),
        pltpu.VMEM((NB2 * STG,), f32),
        pltpu.VMEM((NB2,), i32),
        pltpu.VMEM((16,), i32),
        pltpu.SMEM((NB2,), i32),
        pltpu.SemaphoreType.DMA,
    ]
    return pl.kernel(_bucket_body, out_type=out_type, mesh=_mesh(),
                     scratch_types=scratch,
                     compiler_params=_sc_params())(src, dst, w)


def _aggr16_body(h_hbm, qsrc, qdst, qw, cnt16_hbm, aggr_hbm,
                 table, acc0, acc1, acc2, acc3, src_a, dst_a, w_a,
                 src_b, dst_b, w_b, cblk_v, sem, semb):
    t = lax.axis_index("s") * 2 + lax.axis_index("c")
    iota = lax.iota(jnp.int32, 16)
    HALF = 2 * SQ * H
    accs = [acc0, acc1, acc2, acc3]

    pltpu.sync_copy(cnt16_hbm.at[pl.ds(t * SH * NT * 16, SH * NT * 16)],
                    cblk_v)

    @pl.loop(0, NPB * H // 16)
    def _(z):
        zf = jnp.zeros((16,), jnp.float32)
        for a in accs:
            a[pl.ds(z * 16, 16)] = zf

    def _rmw16(src_v, dst_v, w_v, m, lo):
        mq = (m + 3) >> 2

        @pl.loop(0, mq)
        def _(i4):
            news = []
            for k in range(4):
                i = i4 * 4 + k
                fi = jnp.full((16,), i, jnp.int32)
                s = plsc.load_gather(src_v, [fi])
                dl = plsc.load_gather(dst_v, [fi])
                ww = plsc.load_gather(w_v, [fi])
                r = plsc.load_gather(table, [(s - lo) * H + iota])
                aj = dl * H + iota
                a = plsc.load_gather(accs[k], [aj])
                news.append((aj, jnp.maximum(a, r * ww)))
            for k in range(4):
                aj, nv = news[k]
                plsc.store_scatter(accs[k], [aj], nv)

    for half in range(2):
        lo = half * 2 * SQ
        pltpu.sync_copy(h_hbm.at[pl.ds(half * HALF, HALF)], table)

        for qq in (2 * half, 2 * half + 1):

            @pl.loop(0, NT // 2)
            def _(wp):
                w0 = wp * 2
                slot0 = w0 * NB2 + t * SH + qq
                slot1 = slot0 + NB2
                cnt0 = jnp.max(cblk_v[pl.ds((qq * NT + w0) * 16, 16)])
                cnt1 = jnp.max(cblk_v[pl.ds((qq * NT + w0 + 1) * 16, 16)])
                m0 = jnp.minimum(cnt0, K)
                m1 = jnp.minimum(cnt1, K)
                ca1 = pltpu.async_copy(qsrc.at[pl.ds(slot0 * CAPQ, K)],
                                       src_a, sem)
                ca2 = pltpu.async_copy(qdst.at[pl.ds(slot0 * CAPQ, K)],
                                       dst_a, sem)
                ca3 = pltpu.async_copy(qw.at[pl.ds(slot0 * CAPQ, K)],
                                       w_a, sem)
                cb1 = pltpu.async_copy(qsrc.at[pl.ds(slot1 * CAPQ, K)],
                                       src_b, semb)
                cb2 = pltpu.async_copy(qdst.at[pl.ds(slot1 * CAPQ, K)],
                                       dst_b, semb)
                cb3 = pltpu.async_copy(qw.at[pl.ds(slot1 * CAPQ, K)],
                                       w_b, semb)
                ca1.wait()
                ca2.wait()
                ca3.wait()
                _rmw16(src_a, dst_a, w_a, m0, lo)
                cb1.wait()
                cb2.wait()
                cb3.wait()
                _rmw16(src_b, dst_b, w_b, m1, lo)

                for slot, cnt in ((slot0, cnt0), (slot1, cnt1)):
                    nche = jnp.maximum(((cnt + (K - 1)) >> 7) - 1, 0)

                    @pl.loop(0, nche)
                    def _(cm):
                        c = cm + 1
                        off = slot * CAPQ + c * K
                        m = jnp.minimum(cnt - c * K, K)
                        c1 = pltpu.async_copy(qsrc.at[pl.ds(off, K)],
                                              src_a, sem)
                        c2 = pltpu.async_copy(qdst.at[pl.ds(off, K)],
                                              dst_a, sem)
                        c3 = pltpu.async_copy(qw.at[pl.ds(off, K)],
                                              w_a, sem)
                        c1.wait()
                        c2.wait()
                        c3.wait()
                        _rmw16(src_a, dst_a, w_a, m, lo)

    @pl.loop(0, NPB * H // 16)
    def _(z):
        sl = pl.ds(z * 16, 16)
        acc0[sl] = jnp.maximum(jnp.maximum(acc0[sl], acc1[sl]),
                               jnp.maximum(acc2[sl], acc3[sl]))

    pltpu.sync_copy(acc0, aggr_hbm.at[pl.ds(t * NPB * H, NPB * H)])


def _aggregate16(h1flat, qsrc, qdst, qw, cnt16):
    f32 = jnp.float32
    i32 = jnp.int32
    scratch = [
        pltpu.VMEM((2 * SQ * H,), f32),
        pltpu.VMEM((NPB * H,), f32),
        pltpu.VMEM((NPB * H,), f32),
        pltpu.VMEM((NPB * H,), f32),
        pltpu.VMEM((NPB * H,), f32),
        pltpu.VMEM((K,), i32),
        pltpu.VMEM((K,), i32),
        pltpu.VMEM((K,), f32),
        pltpu.VMEM((K,), i32),
        pltpu.VMEM((K,), i32),
        pltpu.VMEM((K,), f32),
        pltpu.VMEM((SH * NT * 16,), i32),
        pltpu.SemaphoreType.DMA,
        pltpu.SemaphoreType.DMA,
    ]
    return pl.kernel(_aggr16_body,
                     out_type=jax.ShapeDtypeStruct((NPAD * H,), f32),
                     mesh=_mesh(), scratch_types=scratch,
                     compiler_params=_sc_params())(
                         h1flat, qsrc, qdst, qw, cnt16)


def _aggr_body(h_hbm, qsrc, qdst, qw, cnt16_hbm, aggr_hbm,
               hsh, acc0, acc1, rows_a, idx_a, dst_a, w_a,
               idx_b, dst_b, w_b, cblk_v, sem, semb):
    D = D_OUT
    t = lax.axis_index("s") * 2 + lax.axis_index("c")
    iota = lax.iota(jnp.int32, 16)
    NV = D // 16
    accs = [acc0, acc1]

    pltpu.sync_copy(cnt16_hbm.at[pl.ds(t * SH * NT * 16, SH * NT * 16)],
                    cblk_v)

    @pl.loop(0, NPB * D // 16)
    def _(z):
        zf = jnp.zeros((16,), jnp.float32)
        acc0[pl.ds(z * 16, 16)] = zf
        acc1[pl.ds(z * 16, 16)] = zf

    def _sanitize(idx_v, m, lob):
        for v in range(K // 16):
            lanes = iota + v * 16
            cur = idx_v[pl.ds(v * 16, 16)] - lob
            idx_v[pl.ds(v * 16, 16)] = jnp.where(lanes < m, cur, 0)

    def _rmw(rows, dst_v, w_v, m):
        m2 = (m + 1) >> 1

        @pl.loop(0, m2)
        def _(i2):
            news = []
            for k in range(2):
                i = i2 * 2 + k
                fi = jnp.full((16,), i, jnp.int32)
                dl = plsc.load_gather(dst_v, [fi])
                ww = plsc.load_gather(w_v, [fi])
                base = dl * D
                for j in range(NV):
                    aj = base + (iota + j * 16)
                    a = plsc.load_gather(accs[k], [aj])
                    r = rows[i, pl.ds(j * 16, 16)]
                    news.append((k, aj, jnp.maximum(a, r * ww)))
            for k, aj, nv in news:
                plsc.store_scatter(accs[k], [aj], nv)

    for qq in range(SH):
        plsc.subcore_barrier()

        @pl.when(lax.axis_index("s") == 0)
        def _():
            pltpu.sync_copy(h_hbm.at[pl.ds(qq * SQ, SQ), :], hsh)

        plsc.subcore_barrier()
        lob = jnp.full((16,), qq * SQ, jnp.int32)

        @pl.loop(0, NT // 2)
        def _(wp):
            w0 = wp * 2
            slot0 = w0 * NB2 + t * SH + qq
            slot1 = slot0 + NB2
            cnt0 = jnp.max(cblk_v[pl.ds((qq * NT + w0) * 16, 16)])
            cnt1 = jnp.max(cblk_v[pl.ds((qq * NT + w0 + 1) * 16, 16)])
            m0 = jnp.minimum(cnt0, K)
            m1 = jnp.minimum(cnt1, K)
            ca1 = pltpu.async_copy(qsrc.at[pl.ds(slot0 * CAPQ, K)],
                                   idx_a, sem)
            ca2 = pltpu.async_copy(qdst.at[pl.ds(slot0 * CAPQ, K)],
                                   dst_a, sem)
            ca3 = pltpu.async_copy(qw.at[pl.ds(slot0 * CAPQ, K)], w_a, sem)
            cb1 = pltpu.async_copy(qsrc.at[pl.ds(slot1 * CAPQ, K)],
                                   idx_b, semb)
            cb2 = pltpu.async_copy(qdst.at[pl.ds(slot1 * CAPQ, K)],
                                   dst_b, semb)
            cb3 = pltpu.async_copy(qw.at[pl.ds(slot1 * CAPQ, K)], w_b, semb)
            ca1.wait()
            _sanitize(idx_a, m0, lob)
            pltpu.sync_copy(hsh.at[idx_a], rows_a)
            ca2.wait()
            ca3.wait()
            _rmw(rows_a, dst_a, w_a, m0)
            cb1.wait()
            _sanitize(idx_b, m1, lob)
            pltpu.sync_copy(hsh.at[idx_b], rows_a)
            cb2.wait()
            cb3.wait()
            _rmw(rows_a, dst_b, w_b, m1)

            for slot, cnt in ((slot0, cnt0), (slot1, cnt1)):
                nche = jnp.maximum(((cnt + (K - 1)) >> 7) - 1, 0)

                @pl.loop(0, nche)
                def _(cm):
                    c = cm + 1
                    off = slot * CAPQ + c * K
                    m = jnp.minimum(cnt - c * K, K)
                    c1 = pltpu.async_copy(qsrc.at[pl.ds(off, K)], idx_a, sem)
                    c2 = pltpu.async_copy(qdst.at[pl.ds(off, K)], dst_a, sem)
                    c3 = pltpu.async_copy(qw.at[pl.ds(off, K)], w_a, sem)
                    c1.wait()
                    _sanitize(idx_a, m, lob)
                    pltpu.sync_copy(hsh.at[idx_a], rows_a)
                    c2.wait()
                    c3.wait()
                    _rmw(rows_a, dst_a, w_a, m)

    @pl.loop(0, NPB * D // 16)
    def _(z):
        sl = pl.ds(z * 16, 16)
        acc0[sl] = jnp.maximum(acc0[sl], acc1[sl])

    pltpu.sync_copy(acc0, aggr_hbm.at[pl.ds(t * NPB * D, NPB * D)])


def _aggregate(h, qsrc, qdst, qw, cnt16):
    f32 = jnp.float32
    D = D_OUT
    i32 = jnp.int32
    scratch = [
        pltpu.VMEM_SHARED((SQ, D), f32),
        pltpu.VMEM((NPB * D,), f32),
        pltpu.VMEM((NPB * D,), f32),
        pltpu.VMEM((K, D), f32),
        pltpu.VMEM((K,), i32),
        pltpu.VMEM((K,), i32),
        pltpu.VMEM((K,), f32),
        pltpu.VMEM((K,), i32),
        pltpu.VMEM((K,), i32),
        pltpu.VMEM((K,), f32),
        pltpu.VMEM((SH * NT * 16,), i32),
        pltpu.SemaphoreType.DMA,
        pltpu.SemaphoreType.DMA,
    ]
    return pl.kernel(_aggr_body,
                     out_type=jax.ShapeDtypeStruct((NPAD * D,), f32),
                     mesh=_mesh(), scratch_types=scratch,
                     compiler_params=_sc_params())(
                         h, qsrc, qdst, qw, cnt16)


def _mm1_body(x_ref, wt_ref, b_ref, o_ref):
    o_ref[...] = lax.dot_general(
        x_ref[...], wt_ref[...], (((1,), (0,)), ((), ())),
        preferred_element_type=jnp.float32) + b_ref[...]


def _mm1(x, w1t, b1):
    grid = N // ROWBLK
    return pl.pallas_call(
        _mm1_body,
        grid=(grid,),
        in_specs=[
            pl.BlockSpec((ROWBLK, D_IN), lambda i: (i, 0)),
            pl.BlockSpec((D_IN, H), lambda i: (0, 0)),
            pl.BlockSpec((1, H), lambda i: (0, 0)),
        ],
        out_specs=pl.BlockSpec((ROWBLK, H), lambda i: (i, 0)),
        out_shape=jax.ShapeDtypeStruct((N, H), jnp.float32),
    )(x, w1t, b1)


def _mid_body(h1_ref, a1_ref, wt_ref, b_ref, o_ref):
    x2 = jnp.maximum(h1_ref[...] - a1_ref[...], 0.0)
    o_ref[...] = lax.dot_general(
        x2, wt_ref[...], (((1,), (0,)), ((), ())),
        preferred_element_type=jnp.float32) + b_ref[...]


def _mid(h1, a1, w2t, b2):
    grid = N // ROWBLK
    return pl.pallas_call(
        _mid_body,
        grid=(grid,),
        in_specs=[
            pl.BlockSpec((ROWBLK, H), lambda i: (i, 0)),
            pl.BlockSpec((ROWBLK, H), lambda i: (i, 0)),
            pl.BlockSpec((H, D_OUT), lambda i: (0, 0)),
            pl.BlockSpec((1, D_OUT), lambda i: (0, 0)),
        ],
        out_specs=pl.BlockSpec((ROWBLK, D_OUT), lambda i: (i, 0)),
        out_shape=jax.ShapeDtypeStruct((N, D_OUT), jnp.float32),
    )(h1, a1, w2t, b2)


def _final_body(h2_ref, a2_ref, o_ref):
    z = h2_ref[...] - a2_ref[...]
    zmax = jnp.max(z, axis=1, keepdims=True)
    ez = jnp.exp(z - zmax)
    s = jnp.sum(ez, axis=1, keepdims=True)
    o_ref[...] = z - zmax - jnp.log(s)


def _final(h2, a2):
    grid = N // ROWBLK
    return pl.pallas_call(
        _final_body,
        grid=(grid,),
        in_specs=[
            pl.BlockSpec((ROWBLK, D_OUT), lambda i: (i, 0)),
            pl.BlockSpec((ROWBLK, D_OUT), lambda i: (i, 0)),
        ],
        out_specs=pl.BlockSpec((ROWBLK, D_OUT), lambda i: (i, 0)),
        out_shape=jax.ShapeDtypeStruct((N, D_OUT), jnp.float32),
    )(h2, a2)


def kernel(x, edge_index, edge_weight, W1, b1, W2, b2):
    src = edge_index[0]
    dst = edge_index[1]
    qsrc, qdst, qw, cnt16 = _bucket(src, dst, edge_weight)
    h1 = _mm1(x, W1.T, b1.reshape(1, H))
    h1p = jnp.pad(h1, ((0, SH * SQ - N), (0, 0)))
    a1 = _aggregate16(h1p.reshape(SH * SQ * H), qsrc, qdst, qw, cnt16)
    a1 = a1.reshape(NPAD, H)[:N]
    h2 = _mid(h1, a1, W2.T, b2.reshape(1, D_OUT))
    h2p = jnp.pad(h2, ((0, SH * SQ - N), (0, 0)))
    a2 = _aggregate(h2p, qsrc, qdst, qw, cnt16)
    a2 = a2.reshape(NPAD, D_OUT)[:N]
    return _final(h2, a2)
